# Initial kernel scaffold; baseline (speedup 1.0000x reference)
import jax
import jax.numpy as jnp
from jax.experimental import pallas as pl


def kernel(x, edge_index, edge_label_index, W1l, W1r, b1, gamma, beta, rm, rv, W2l, W2r, b2):
    raise NotImplementedError("write your pallas kernel here")


# speedup vs baseline: 6.1783x; 6.1783x over previous
import dataclasses

import jax
import jax.numpy as jnp
from jax import lax
from jax.experimental import pallas as pl
from jax.experimental.pallas import tpu as pltpu
from jax.experimental.pallas import tpu_sc as plsc

N_NODES = 10000
EPS = 1e-5
NC, NS = 2, 16
NW = NC * NS
CHUNK = 128
N_PAD = 10240
RPT = N_PAD // NS

_SC_PARAMS = dataclasses.replace(pltpu.CompilerParams(),
                                 needs_layout_passes=False)


def _seg_sum_kernel(num_edges: int, feat: int, with_hist: bool):
  mesh = plsc.VectorSubcoreMesh(
      core_axis_name="c", subcore_axis_name="s", num_cores=NC, num_subcores=NS)
  out_type = [jax.ShapeDtypeStruct((NC, N_PAD, feat), jnp.float32)]
  if with_hist:
    out_type.append(jax.ShapeDtypeStruct((NW * N_PAD,), jnp.float32))
  scratch = [
      pltpu.VMEM((CHUNK,), jnp.int32),
      pltpu.VMEM((CHUNK,), jnp.int32),
      pltpu.VMEM((CHUNK, feat), jnp.float32),
      pltpu.VMEM_SHARED((N_PAD, feat), jnp.float32),
      pltpu.SemaphoreType.DMA,
  ]
  if with_hist:
    scratch.insert(3, pltpu.VMEM((N_PAD,), jnp.float32))
  num_rows = num_edges // CHUNK

  def body(src_hbm, dst_hbm, data_hbm, zeros_hbm, *rest):
    if with_hist:
      p_out, h_out, sidx, didx, rows, hist, acc, sem = rest
    else:
      p_out, sidx, didx, rows, acc, sem = rest
      hist = h_out = None
    c = lax.axis_index("c")
    s = lax.axis_index("s")
    wid = s * NC + c

    pltpu.sync_copy(zeros_hbm.at[pl.ds(s * RPT, RPT)],
                    acc.at[pl.ds(s * RPT, RPT)])
    if with_hist:
      @pl.loop(0, N_PAD // 16)
      def _(i):
        hist[pl.ds(i * 16, 16)] = jnp.zeros((16,), jnp.float32)
    plsc.subcore_barrier()

    ones16 = jnp.ones((16,), jnp.float32)

    @pl.loop(wid, num_rows, step=NW)
    def _(j):
      pltpu.sync_copy(src_hbm.at[pl.ds(j * CHUNK, CHUNK)], sidx)
      pltpu.async_copy(data_hbm.at[sidx], rows, sem).wait()
      pltpu.sync_copy(dst_hbm.at[pl.ds(j * CHUNK, CHUNK)], didx)
      pltpu.sync_copy(rows, acc.at[didx], add=True)
      if with_hist:
        for q in range(CHUNK // 16):
          iv = didx[pl.ds(q * 16, 16)]
          plsc.addupdate_scatter(hist, [iv], ones16)

    plsc.subcore_barrier()
    pltpu.sync_copy(acc.at[pl.ds(s * RPT, RPT)],
                    p_out.at[c, pl.ds(s * RPT, RPT)])
    if with_hist:
      pltpu.sync_copy(hist, h_out.at[pl.ds(wid * N_PAD, N_PAD)])

  return pl.kernel(body, out_type=tuple(out_type), mesh=mesh,
                   scratch_types=scratch, compiler_params=_SC_PARAMS)


def _decode_kernel(num_pairs: int, feat: int):
  mesh = plsc.VectorSubcoreMesh(
      core_axis_name="c", subcore_axis_name="s", num_cores=NC, num_subcores=NS)
  scratch = [
      pltpu.VMEM((CHUNK,), jnp.int32),
      pltpu.VMEM((CHUNK,), jnp.int32),
      pltpu.VMEM((CHUNK, feat), jnp.float32),
      pltpu.VMEM((CHUNK, feat), jnp.float32),
      pltpu.SemaphoreType.DMA,
      pltpu.SemaphoreType.DMA,
  ]
  num_rows = num_pairs // CHUNK

  def body(a_hbm, b_hbm, z_hbm, out_hbm, aidx, bidx, arows, brows, sema, semb):
    c = lax.axis_index("c")
    s = lax.axis_index("s")
    wid = s * NC + c

    @pl.loop(wid, num_rows, step=NW)
    def _(j):
      pltpu.sync_copy(a_hbm.at[pl.ds(j * CHUNK, CHUNK)], aidx)
      pltpu.sync_copy(b_hbm.at[pl.ds(j * CHUNK, CHUNK)], bidx)
      cpa = pltpu.async_copy(z_hbm.at[aidx], arows, sema)
      cpb = pltpu.async_copy(z_hbm.at[bidx], brows, semb)
      cpa.wait()
      cpb.wait()

      @pl.loop(0, CHUNK)
      def _(i):
        for q in range(64 // 16):
          sl = pl.ds(q * 16, 16)
          arows[i, sl] = arows[i, sl] * brows[i, sl]

      pltpu.sync_copy(arows, out_hbm.at[pl.ds(j * CHUNK, CHUNK)])

  return pl.kernel(
      body,
      out_type=jax.ShapeDtypeStruct((num_pairs, feat), jnp.float32),
      mesh=mesh, scratch_types=scratch)


def _tc_layer1(p, hist, x, W1l_s, W1r_s, bias1, W2r, b2):
  def body(p_ref, h_ref, x_ref, wl_ref, wr_ref, b1_ref, w2r_ref,
           b2_ref, hout_ref, r2_ref, invc_ref):
    ones = jnp.ones((NW, 1), jnp.float32)
    hists = h_ref[:, :N_NODES]
    cnt = lax.dot_general(hists, ones, (((0,), (0,)), ((), ())),
                          preferred_element_type=jnp.float32)
    invc = 1.0 / jnp.maximum(cnt, 1.0)
    agg = (p_ref[0, :N_NODES] + p_ref[1, :N_NODES]) * invc
    h = agg @ wl_ref[...].T + x_ref[...] @ wr_ref[...].T + b1_ref[...]
    h = jnp.maximum(h, 0.0)
    hout_ref[...] = h
    r2_ref[...] = h @ w2r_ref[...].T + b2_ref[...]
    invc_ref[...] = invc

  return pl.pallas_call(
      body,
      out_shape=(
          jax.ShapeDtypeStruct((N_NODES, 128), jnp.float32),
          jax.ShapeDtypeStruct((N_NODES, 64), jnp.float32),
          jax.ShapeDtypeStruct((N_NODES, 1), jnp.float32),
      ),
  )(p, hist, x, W1l_s, W1r_s, bias1, W2r, b2)


def _tc_layer2(q, invc, r2, W2l):
  def body(q_ref, invc_ref, r2_ref, w2l_ref, z_ref):
    agg = (q_ref[0, :N_NODES] + q_ref[1, :N_NODES]) * invc_ref[...]
    z_ref[:, :64] = agg @ w2l_ref[...].T + r2_ref[...]
    z_ref[:, 64:] = jnp.zeros((N_NODES, 64), jnp.float32)

  return pl.pallas_call(
      body,
      out_shape=jax.ShapeDtypeStruct((N_NODES, 128), jnp.float32),
  )(q, invc, r2, W2l)


def _tc_rowsum(prod):
  def body(p_ref, o_ref):
    o_ref[...] = jnp.sum(p_ref[...], axis=-1, keepdims=True)

  rows = prod.shape[0]
  blk = rows // 8
  out = pl.pallas_call(
      body,
      grid=(8,),
      in_specs=[pl.BlockSpec((blk, prod.shape[1]), lambda i: (i, 0))],
      out_specs=pl.BlockSpec((blk, 1), lambda i: (i, 0)),
      out_shape=jax.ShapeDtypeStruct((rows, 1), jnp.float32),
  )(prod)
  return out.reshape(rows)


def kernel(x, edge_index, edge_label_index, W1l, W1r, b1, gamma, beta, rm, rv,
           W2l, W2r, b2):
  E = edge_index.shape[1]
  EL = edge_label_index.shape[1]

  ei = edge_index.astype(jnp.int32)
  src, dst = ei[0], ei[1]
  eli = edge_label_index.astype(jnp.int32)
  a_idx, b_idx = eli[0], eli[1]

  z128 = jnp.zeros((N_PAD, 128), jnp.float32)

  scale = gamma / jnp.sqrt(rv + EPS)
  W1l_s = W1l * scale[:, None]
  W1r_s = W1r * scale[:, None]
  bias1 = ((b1 - rm) * scale + beta)[None, :]

  p, hist = _seg_sum_kernel(E, 128, True)(src, dst, x, z128)
  h, r2, invc = _tc_layer1(p, hist.reshape(NW, N_PAD), x, W1l_s, W1r_s, bias1,
                           W2r, b2[None, :])
  (q,) = _seg_sum_kernel(E, 128, False)(src, dst, h, z128)
  z = _tc_layer2(q, invc, r2, W2l)
  prod = _decode_kernel(EL, 128)(a_idx, b_idx, z)
  return _tc_rowsum(prod)
